# baseline (speedup 1.0000x reference)
import jax
import jax.numpy as jnp
from jax import lax
from jax.experimental import pallas as pl
from jax.experimental.pallas import tpu as pltpu
from jax.experimental.pallas import tpu_sc as plsc

D = 128
SCALE = float(D) ** 0.5
NW = 32
CHUNK = 50
LANES = 16
NBUF = 8


def _body(x_hbm, table_hbm, pos_hbm, out_hbm, idx_v, pos_v, refs):
    ins = refs[:NBUF]
    outs = refs[NBUF:NBUF + 2]
    gsems = refs[NBUF + 2:2 * NBUF + 2]
    ssems = refs[2 * NBUF + 2:]
    n_chunks_w = idx_v.shape[0]
    seq = pos_v.shape[0]
    cpseq = seq // CHUNK
    wid = lax.axis_index("s") * 2 + lax.axis_index("c")
    gbase = wid * (n_chunks_w * CHUNK)

    pltpu.sync_copy(x_hbm.at[pl.ds(wid * n_chunks_w, n_chunks_w)], idx_v)
    pltpu.sync_copy(pos_hbm, pos_v)

    for b in range(NBUF):
        pltpu.async_copy(table_hbm.at[idx_v.at[b]], ins[b], gsems[b])

    n_iters = n_chunks_w // NBUF

    def iter_body(t, carry):
        for p in range(NBUF):
            j = NBUF * t + p
            inb, gs = ins[p], gsems[p]
            outb, ss = outs[p % 2], ssems[p % 2]
            pltpu.make_async_copy(table_hbm.at[idx_v.at[j]], inb, gs).wait()

            @pl.when(j > 0)
            def _prev_scatter():
                pltpu.async_copy(
                    outs[(p + 1) % 2],
                    out_hbm.at[pl.ds(gbase + (j - 1) * CHUNK, CHUNK)],
                    ssems[(p + 1) % 2])

            @pl.when(j > 1)
            def _wait_prev_scatter():
                pltpu.make_async_copy(
                    outb, out_hbm.at[pl.ds(gbase + (j - 2) * CHUNK, CHUNK)],
                    ss).wait()

            poff = (p % cpseq) * CHUNK

            @plsc.parallel_loop(0, CHUNK, step=1, unroll=4)
            def _compute(r):
                for c in range(D // LANES):
                    sl = pl.ds(c * LANES, LANES)
                    outb[r, sl] = inb[r, sl] * SCALE + pos_v[poff + r, sl]

            @pl.when(t < n_iters - 1)
            def _next_gather():
                pltpu.async_copy(table_hbm.at[idx_v.at[j + NBUF]], inb, gs)
        return carry

    lax.fori_loop(0, n_iters, iter_body, 0)

    last = n_chunks_w - 1
    pltpu.make_async_copy(
        outs[(last - 1) % 2],
        out_hbm.at[pl.ds(gbase + (last - 1) * CHUNK, CHUNK)],
        ssems[(last - 1) % 2]).wait()
    pltpu.async_copy(
        outs[last % 2], out_hbm.at[pl.ds(gbase + last * CHUNK, CHUNK)],
        ssems[last % 2])
    pltpu.make_async_copy(
        outs[last % 2], out_hbm.at[pl.ds(gbase + last * CHUNK, CHUNK)],
        ssems[last % 2]).wait()


def kernel(x, table, pos_encoding):
    B, S = x.shape
    N = B * S
    n_chunks = N // CHUNK
    x2 = x.reshape(n_chunks, CHUNK)
    seq = pos_encoding.shape[0]

    mesh = plsc.VectorSubcoreMesh(core_axis_name="c", subcore_axis_name="s")

    def body(x_hbm, table_hbm, pos_hbm, out_hbm, idx_v, pos_v, *refs):
        _body(x_hbm, table_hbm, pos_hbm, out_hbm, idx_v, pos_v, refs)

    run = pl.kernel(
        body,
        out_type=jax.ShapeDtypeStruct((N, D), jnp.float32),
        mesh=mesh,
        compiler_params=pltpu.CompilerParams(use_tc_tiling_on_sc=False),
        scratch_types=(
            [pltpu.VMEM((n_chunks // NW, CHUNK), jnp.int32),
             pltpu.VMEM((seq, D), jnp.float32)]
            + [pltpu.VMEM((CHUNK, D), jnp.float32)
               for _ in range(NBUF)]
            + [pltpu.VMEM((CHUNK, D), jnp.float32)
               for _ in range(2)]
            + [pltpu.SemaphoreType.DMA for _ in range(NBUF + 2)]
        ),
    )
    out = run(x2, table, pos_encoding)
    return out.reshape(B, S, D)


# speedup vs baseline: 1.0735x; 1.0735x over previous
import jax
import jax.numpy as jnp
from jax import lax
from jax.experimental import pallas as pl
from jax.experimental.pallas import tpu as pltpu
from jax.experimental.pallas import tpu_sc as plsc

D = 128
SCALE = float(D) ** 0.5
NW = 32
CHUNK = 100
LANES = 16
NBUF = 4


def _body(x_hbm, table_hbm, pos_hbm, out_hbm, idx_v, pos_v, refs):
    ins = refs[:NBUF]
    outs = refs[NBUF:NBUF + 2]
    gsems = refs[NBUF + 2:2 * NBUF + 2]
    ssems = refs[2 * NBUF + 2:]
    n_chunks_w = idx_v.shape[0]
    seq = pos_v.shape[0]
    cpseq = seq // CHUNK
    wid = lax.axis_index("s") * 2 + lax.axis_index("c")
    gbase = wid * (n_chunks_w * CHUNK)

    pltpu.sync_copy(x_hbm.at[pl.ds(wid * n_chunks_w, n_chunks_w)], idx_v)
    pltpu.sync_copy(pos_hbm, pos_v)

    for b in range(NBUF):
        pltpu.async_copy(table_hbm.at[idx_v.at[b]], ins[b], gsems[b])

    n_iters = n_chunks_w // NBUF

    def iter_body(t, carry):
        for p in range(NBUF):
            j = NBUF * t + p
            inb, gs = ins[p], gsems[p]
            outb, ss = outs[p % 2], ssems[p % 2]
            pltpu.make_async_copy(table_hbm.at[idx_v.at[j]], inb, gs).wait()

            @pl.when(j > 0)
            def _prev_scatter():
                pltpu.async_copy(
                    outs[(p + 1) % 2],
                    out_hbm.at[pl.ds(gbase + (j - 1) * CHUNK, CHUNK)],
                    ssems[(p + 1) % 2])

            @pl.when(j > 1)
            def _wait_prev_scatter():
                pltpu.make_async_copy(
                    outb, out_hbm.at[pl.ds(gbase + (j - 2) * CHUNK, CHUNK)],
                    ss).wait()

            poff = (p % cpseq) * CHUNK

            @plsc.parallel_loop(0, CHUNK, step=1, unroll=4)
            def _compute(r):
                for c in range(D // LANES):
                    sl = pl.ds(c * LANES, LANES)
                    outb[r, sl] = inb[r, sl] * SCALE + pos_v[poff + r, sl]

            @pl.when(t < n_iters - 1)
            def _next_gather():
                pltpu.async_copy(table_hbm.at[idx_v.at[j + NBUF]], inb, gs)
        return carry

    lax.fori_loop(0, n_iters, iter_body, 0)

    last = n_chunks_w - 1
    pltpu.make_async_copy(
        outs[(last - 1) % 2],
        out_hbm.at[pl.ds(gbase + (last - 1) * CHUNK, CHUNK)],
        ssems[(last - 1) % 2]).wait()
    pltpu.async_copy(
        outs[last % 2], out_hbm.at[pl.ds(gbase + last * CHUNK, CHUNK)],
        ssems[last % 2])
    pltpu.make_async_copy(
        outs[last % 2], out_hbm.at[pl.ds(gbase + last * CHUNK, CHUNK)],
        ssems[last % 2]).wait()


def kernel(x, table, pos_encoding):
    B, S = x.shape
    N = B * S
    n_chunks = N // CHUNK
    x2 = x.reshape(n_chunks, CHUNK)
    seq = pos_encoding.shape[0]

    mesh = plsc.VectorSubcoreMesh(core_axis_name="c", subcore_axis_name="s")

    def body(x_hbm, table_hbm, pos_hbm, out_hbm, idx_v, pos_v, *refs):
        _body(x_hbm, table_hbm, pos_hbm, out_hbm, idx_v, pos_v, refs)

    run = pl.kernel(
        body,
        out_type=jax.ShapeDtypeStruct((N, D), jnp.float32),
        mesh=mesh,
        compiler_params=pltpu.CompilerParams(use_tc_tiling_on_sc=False),
        scratch_types=(
            [pltpu.VMEM((n_chunks // NW, CHUNK), jnp.int32),
             pltpu.VMEM((seq, D), jnp.float32)]
            + [pltpu.VMEM((CHUNK, D), jnp.float32)
               for _ in range(NBUF)]
            + [pltpu.VMEM((CHUNK, D), jnp.float32)
               for _ in range(2)]
            + [pltpu.SemaphoreType.DMA for _ in range(NBUF + 2)]
        ),
    )
    out = run(x2, table, pos_encoding)
    return out.reshape(B, S, D)
